# initial kernel scaffold (unmeasured)
import jax
import jax.numpy as jnp
from jax import lax
from jax.experimental import pallas as pl
from jax.experimental.pallas import tpu as pltpu


def kernel(
    x,
):
    def body(*refs):
        pass

    out_shape = jax.ShapeDtypeStruct(..., jnp.float32)
    return pl.pallas_call(body, out_shape=out_shape)(...)


# baseline (device time: 128293 ns/iter reference)
import jax
import jax.numpy as jnp
from jax import lax
from jax.experimental import pallas as pl
from jax.experimental.pallas import tpu as pltpu

M = 1024
N = 1024
N_STAGES = 4


def kernel(x):
    x2 = x.reshape(M, N)

    def body(x_ref, out_ref, recv_ref, send_sems, recv_sems):
        mx = lax.axis_index("x")
        my = lax.axis_index("y")
        mz = lax.axis_index("z")

        partners = [
            (1 - mx, my, mz),
            (mx, 1 - my, mz),
            (mx, my, mz ^ 1),
            (mx, my, mz ^ 2),
        ]

        barrier = pltpu.get_barrier_semaphore()
        for p in partners:
            pl.semaphore_signal(
                barrier, inc=1, device_id=p, device_id_type=pl.DeviceIdType.MESH
            )
        pl.semaphore_wait(barrier, N_STAGES)

        out_ref[...] = x_ref[...].astype(jnp.bfloat16)

        for s, p in enumerate(partners):
            rdma = pltpu.make_async_remote_copy(
                src_ref=out_ref,
                dst_ref=recv_ref.at[s],
                send_sem=send_sems.at[s],
                recv_sem=recv_sems.at[s],
                device_id=p,
                device_id_type=pl.DeviceIdType.MESH,
            )
            rdma.start()
            rdma.wait()
            out_ref[...] = out_ref[...] + recv_ref[s]

    return pl.pallas_call(
        body,
        out_shape=jax.ShapeDtypeStruct((M, N), jnp.bfloat16),
        in_specs=[pl.BlockSpec(memory_space=pltpu.VMEM)],
        out_specs=pl.BlockSpec(memory_space=pltpu.VMEM),
        scratch_shapes=[
            pltpu.VMEM((N_STAGES, M, N), jnp.bfloat16),
            pltpu.SemaphoreType.DMA((N_STAGES,)),
            pltpu.SemaphoreType.DMA((N_STAGES,)),
        ],
        compiler_params=pltpu.CompilerParams(collective_id=0),
    )(x2)


# device time: 65206 ns/iter; 1.9675x vs baseline; 1.9675x over previous
import jax
import jax.numpy as jnp
from jax import lax
from jax.experimental import pallas as pl
from jax.experimental.pallas import tpu as pltpu

M = 1024
N = 1024
SIZES = (512, 256, 128, 64)


def kernel(x):
    x2 = x.reshape(M, N)

    def body(x_ref, out_ref, rbuf0, rbuf1, rbuf2, rbuf3, send_sems, recv_sems):
        mx = lax.axis_index("x")
        my = lax.axis_index("y")
        mz = lax.axis_index("z")

        stages = [
            ((1 - mx, my, mz), mx, SIZES[0]),
            ((mx, 1 - my, mz), my, SIZES[1]),
            ((mx, my, mz ^ 1), mz & 1, SIZES[2]),
            ((mx, my, mz ^ 2), (mz >> 1) & 1, SIZES[3]),
        ]
        rbufs = [rbuf0, rbuf1, rbuf2, rbuf3]

        barrier = pltpu.get_barrier_semaphore()
        for p, _, _ in stages:
            pl.semaphore_signal(
                barrier, inc=1, device_id=p, device_id_type=pl.DeviceIdType.MESH
            )
        pl.semaphore_wait(barrier, len(stages))

        out_ref[...] = x_ref[...].astype(jnp.bfloat16)

        base = 0
        for s, (p, b, sz) in enumerate(stages):
            send_off = base + (1 - b) * sz
            keep_off = base + b * sz
            rdma = pltpu.make_async_remote_copy(
                src_ref=out_ref.at[pl.ds(send_off, sz)],
                dst_ref=rbufs[s],
                send_sem=send_sems.at[s],
                recv_sem=recv_sems.at[s],
                device_id=p,
                device_id_type=pl.DeviceIdType.MESH,
            )
            rdma.start()
            rdma.wait()
            out_ref[pl.ds(keep_off, sz), :] = (
                out_ref[pl.ds(keep_off, sz), :] + rbufs[s][...]
            )
            base = keep_off

        cur = SIZES[-1]
        for s in range(len(stages) - 1, -1, -1):
            p, b, _ = stages[s]
            idx = len(stages) + (len(stages) - 1 - s)
            rdma = pltpu.make_async_remote_copy(
                src_ref=out_ref.at[pl.ds(base, cur)],
                dst_ref=out_ref.at[pl.ds(base, cur)],
                send_sem=send_sems.at[idx],
                recv_sem=recv_sems.at[idx],
                device_id=p,
                device_id_type=pl.DeviceIdType.MESH,
            )
            rdma.start()
            rdma.wait()
            base = base - b * cur
            cur *= 2

    return pl.pallas_call(
        body,
        out_shape=jax.ShapeDtypeStruct((M, N), jnp.bfloat16),
        in_specs=[pl.BlockSpec(memory_space=pltpu.VMEM)],
        out_specs=pl.BlockSpec(memory_space=pltpu.VMEM),
        scratch_shapes=[
            pltpu.VMEM((SIZES[0], N), jnp.bfloat16),
            pltpu.VMEM((SIZES[1], N), jnp.bfloat16),
            pltpu.VMEM((SIZES[2], N), jnp.bfloat16),
            pltpu.VMEM((SIZES[3], N), jnp.bfloat16),
            pltpu.SemaphoreType.DMA((8,)),
            pltpu.SemaphoreType.DMA((8,)),
        ],
        compiler_params=pltpu.CompilerParams(collective_id=0),
    )(x2)


# device time: 51219 ns/iter; 2.5048x vs baseline; 1.2731x over previous
import jax
import jax.numpy as jnp
from jax import lax
from jax.experimental import pallas as pl
from jax.experimental.pallas import tpu as pltpu

M = 1024
N = 1024
NCOL = N // 2
SIZES = (512, 256, 128, 64)
RBUF_OFFS = (0, 512, 768, 896)
N_STAGES = 4


def kernel(x):
    x2 = x.reshape(M, N)

    def body(x_ref, out_ref, rbuf_a, rbuf_b, send_sems, recv_sems):
        mx = lax.axis_index("x")
        my = lax.axis_index("y")
        mz = lax.axis_index("z")

        dim_x = ((1 - mx, my, mz), mx)
        dim_y = ((mx, 1 - my, mz), my)
        dim_z1 = ((mx, my, mz ^ 1), mz & 1)
        dim_z2 = ((mx, my, mz ^ 2), (mz >> 1) & 1)

        streams = [
            ([dim_x, dim_y, dim_z1, dim_z2], 0, rbuf_a, 0),
            ([dim_z1, dim_z2, dim_x, dim_y], NCOL, rbuf_b, 2 * N_STAGES),
        ]

        barrier = pltpu.get_barrier_semaphore()
        for p, _ in streams[0][0]:
            pl.semaphore_signal(
                barrier, inc=1, device_id=p, device_id_type=pl.DeviceIdType.MESH
            )
        pl.semaphore_wait(barrier, N_STAGES)

        out_ref[...] = x_ref[...].astype(jnp.bfloat16)

        bases = [0, 0]
        for s in range(N_STAGES):
            sz = SIZES[s]
            rdmas = []
            for dims, col, rbuf, sem0 in streams:
                p, b = dims[s]
                st = bases[0] if col == 0 else bases[1]
                rdma = pltpu.make_async_remote_copy(
                    src_ref=out_ref.at[
                        pl.ds(st + (1 - b) * sz, sz), pl.ds(col, NCOL)
                    ],
                    dst_ref=rbuf.at[pl.ds(RBUF_OFFS[s], sz)],
                    send_sem=send_sems.at[sem0 + s],
                    recv_sem=recv_sems.at[sem0 + s],
                    device_id=p,
                    device_id_type=pl.DeviceIdType.MESH,
                )
                rdma.start()
                rdmas.append(rdma)
            for i, (dims, col, rbuf, _) in enumerate(streams):
                p, b = dims[s]
                rdmas[i].wait()
                keep = bases[i] + b * sz
                out_ref[pl.ds(keep, sz), pl.ds(col, NCOL)] = (
                    out_ref[pl.ds(keep, sz), pl.ds(col, NCOL)]
                    + rbuf[pl.ds(RBUF_OFFS[s], sz), :]
                )
                bases[i] = keep

        cur = SIZES[-1]
        for s in range(N_STAGES - 1, -1, -1):
            rdmas = []
            for i, (dims, col, _, sem0) in enumerate(streams):
                p, b = dims[s]
                idx = sem0 + N_STAGES + (N_STAGES - 1 - s)
                rdma = pltpu.make_async_remote_copy(
                    src_ref=out_ref.at[pl.ds(bases[i], cur), pl.ds(col, NCOL)],
                    dst_ref=out_ref.at[pl.ds(bases[i], cur), pl.ds(col, NCOL)],
                    send_sem=send_sems.at[idx],
                    recv_sem=recv_sems.at[idx],
                    device_id=p,
                    device_id_type=pl.DeviceIdType.MESH,
                )
                rdma.start()
                rdmas.append(rdma)
            for i, (dims, _, _, _) in enumerate(streams):
                _, b = dims[s]
                rdmas[i].wait()
                bases[i] = bases[i] - b * cur
            cur *= 2

    return pl.pallas_call(
        body,
        out_shape=jax.ShapeDtypeStruct((M, N), jnp.bfloat16),
        in_specs=[pl.BlockSpec(memory_space=pltpu.VMEM)],
        out_specs=pl.BlockSpec(memory_space=pltpu.VMEM),
        scratch_shapes=[
            pltpu.VMEM((960, NCOL), jnp.bfloat16),
            pltpu.VMEM((960, NCOL), jnp.bfloat16),
            pltpu.SemaphoreType.DMA((4 * N_STAGES,)),
            pltpu.SemaphoreType.DMA((4 * N_STAGES,)),
        ],
        compiler_params=pltpu.CompilerParams(collective_id=0),
    )(x2)


# device time: 46155 ns/iter; 2.7796x vs baseline; 1.1097x over previous
import jax
import jax.numpy as jnp
from jax import lax
from jax.experimental import pallas as pl
from jax.experimental.pallas import tpu as pltpu

M = 1024
N = 1024
NCOL = N // 2


def kernel(x):
    x2 = x.reshape(M, N)

    def body(x_ref, out_ref, rbuf_a, rbuf_b, send_sems, recv_sems):
        mx = lax.axis_index("x")
        my = lax.axis_index("y")
        mz = lax.axis_index("z")

        fme = 2 * mx + my
        xy_targets = [
            ((1 - mx, my, mz), 2 * (1 - mx) + my),
            ((mx, 1 - my, mz), 2 * mx + (1 - my)),
            ((1 - mx, 1 - my, mz), 2 * (1 - mx) + (1 - my)),
        ]
        z1p, b1 = (mx, my, mz ^ 1), mz & 1
        z2p, b2 = (mx, my, mz ^ 2), (mz >> 1) & 1

        barrier = pltpu.get_barrier_semaphore()
        for p, _ in xy_targets:
            pl.semaphore_signal(
                barrier, inc=1, device_id=p, device_id_type=pl.DeviceIdType.MESH
            )
        for p in (z1p, z2p):
            pl.semaphore_signal(
                barrier, inc=1, device_id=p, device_id_type=pl.DeviceIdType.MESH
            )
        pl.semaphore_wait(barrier, 5)

        out_ref[...] = x_ref[...].astype(jnp.bfloat16)

        def start_xy_rs(base, q, col, roff, sem0, rbuf):
            descs = []
            for j, (tgt, ft) in enumerate(xy_targets):
                d = pltpu.make_async_remote_copy(
                    src_ref=out_ref.at[pl.ds(base + ft * q, q), pl.ds(col, NCOL)],
                    dst_ref=rbuf.at[pl.ds(roff + j * q, q)],
                    send_sem=send_sems.at[sem0 + j],
                    recv_sem=recv_sems.at[sem0 + j],
                    device_id=tgt,
                    device_id_type=pl.DeviceIdType.MESH,
                )
                d.start()
                descs.append(d)
            return descs

        def finish_xy_rs(descs, base, q, col, roff, rbuf):
            for d in descs:
                d.wait()
            keep = base + fme * q
            out_ref[pl.ds(keep, q), pl.ds(col, NCOL)] = (
                out_ref[pl.ds(keep, q), pl.ds(col, NCOL)]
                + rbuf[pl.ds(roff, q), :]
                + rbuf[pl.ds(roff + q, q), :]
                + rbuf[pl.ds(roff + 2 * q, q), :]
            )
            return keep

        def start_z_rs(base, b, sz, col, roff, sem, rbuf, tgt):
            d = pltpu.make_async_remote_copy(
                src_ref=out_ref.at[
                    pl.ds(base + (1 - b) * sz, sz), pl.ds(col, NCOL)
                ],
                dst_ref=rbuf.at[pl.ds(roff, sz)],
                send_sem=send_sems.at[sem],
                recv_sem=recv_sems.at[sem],
                device_id=tgt,
                device_id_type=pl.DeviceIdType.MESH,
            )
            d.start()
            return d

        def finish_z_rs(d, base, b, sz, col, roff, rbuf):
            d.wait()
            keep = base + b * sz
            out_ref[pl.ds(keep, sz), pl.ds(col, NCOL)] = (
                out_ref[pl.ds(keep, sz), pl.ds(col, NCOL)]
                + rbuf[pl.ds(roff, sz), :]
            )
            return keep

        def start_ag(base, cur, col, sem0, tgts):
            descs = []
            for j, tgt in enumerate(tgts):
                d = pltpu.make_async_remote_copy(
                    src_ref=out_ref.at[pl.ds(base, cur), pl.ds(col, NCOL)],
                    dst_ref=out_ref.at[pl.ds(base, cur), pl.ds(col, NCOL)],
                    send_sem=send_sems.at[sem0 + j],
                    recv_sem=recv_sems.at[sem0 + j],
                    device_id=tgt,
                    device_id_type=pl.DeviceIdType.MESH,
                )
                d.start()
                descs.append(d)
            return descs

        def finish_ag(descs):
            for d in descs:
                d.wait()

        xy_tgts_only = [t for t, _ in xy_targets]

        dA = start_xy_rs(0, 256, 0, 0, 0, rbuf_a)
        dB = start_z_rs(0, b1, 512, NCOL, 0, 10, rbuf_b, z1p)
        base_a = finish_xy_rs(dA, 0, 256, 0, 0, rbuf_a)
        base_b = finish_z_rs(dB, 0, b1, 512, NCOL, 0, rbuf_b)

        dA = start_z_rs(base_a, b1, 128, 0, 768, 3, rbuf_a, z1p)
        dB = start_z_rs(base_b, b2, 256, NCOL, 512, 11, rbuf_b, z2p)
        base_a = finish_z_rs(dA, base_a, b1, 128, 0, 768, rbuf_a)
        base_b = finish_z_rs(dB, base_b, b2, 256, NCOL, 512, rbuf_b)

        dA = start_z_rs(base_a, b2, 64, 0, 896, 4, rbuf_a, z2p)
        dB = start_xy_rs(base_b, 64, NCOL, 768, 12, rbuf_b)
        base_a = finish_z_rs(dA, base_a, b2, 64, 0, 896, rbuf_a)
        base_b = finish_xy_rs(dB, base_b, 64, NCOL, 768, rbuf_b)

        dA = start_ag(base_a, 64, 0, 5, [z2p])
        dB = start_ag(base_b, 64, NCOL, 15, xy_tgts_only)
        finish_ag(dA)
        finish_ag(dB)
        base_a = base_a - b2 * 64
        base_b = base_b - fme * 64

        dA = start_ag(base_a, 128, 0, 6, [z1p])
        dB = start_ag(base_b, 256, NCOL, 18, [z2p])
        finish_ag(dA)
        finish_ag(dB)
        base_a = base_a - b1 * 128
        base_b = base_b - b2 * 256

        dA = start_ag(base_a, 256, 0, 7, xy_tgts_only)
        dB = start_ag(base_b, 512, NCOL, 19, [z1p])
        finish_ag(dA)
        finish_ag(dB)

    return pl.pallas_call(
        body,
        out_shape=jax.ShapeDtypeStruct((M, N), jnp.bfloat16),
        in_specs=[pl.BlockSpec(memory_space=pltpu.VMEM)],
        out_specs=pl.BlockSpec(memory_space=pltpu.VMEM),
        scratch_shapes=[
            pltpu.VMEM((960, NCOL), jnp.bfloat16),
            pltpu.VMEM((960, NCOL), jnp.bfloat16),
            pltpu.SemaphoreType.DMA((20,)),
            pltpu.SemaphoreType.DMA((20,)),
        ],
        compiler_params=pltpu.CompilerParams(collective_id=0),
    )(x2)


# device time: 41873 ns/iter; 3.0639x vs baseline; 1.1023x over previous
import jax
import jax.numpy as jnp
from jax import lax
from jax.experimental import pallas as pl
from jax.experimental.pallas import tpu as pltpu

M = 1024
N = 1024
NCOL = N // 2


def kernel(x):
    x2 = x.reshape(M, N)

    def body(x_ref, out_ref, rbuf_a, rbuf_b, send_sems, recv_sems):
        mx = lax.axis_index("x")
        my = lax.axis_index("y")
        mz = lax.axis_index("z")

        fme = 2 * mx + my
        xy_targets = [
            ((1 - mx, my, mz), 2 * (1 - mx) + my),
            ((mx, 1 - my, mz), 2 * mx + (1 - my)),
            ((1 - mx, 1 - my, mz), 2 * (1 - mx) + (1 - my)),
        ]
        xy_tgts_only = [t for t, _ in xy_targets]
        z1p, b1 = (mx, my, mz ^ 1), mz & 1
        z2p = (mx, my, mz ^ 2)

        barrier = pltpu.get_barrier_semaphore()
        for p in xy_tgts_only + [z1p, z2p]:
            pl.semaphore_signal(
                barrier, inc=1, device_id=p, device_id_type=pl.DeviceIdType.MESH
            )
        pl.semaphore_wait(barrier, 5)

        out_ref[...] = x_ref[...].astype(jnp.bfloat16)

        def start_xy_rs(base, q, col, roff, sem0, rbuf):
            descs = []
            for j, (tgt, ft) in enumerate(xy_targets):
                d = pltpu.make_async_remote_copy(
                    src_ref=out_ref.at[pl.ds(base + ft * q, q), pl.ds(col, NCOL)],
                    dst_ref=rbuf.at[pl.ds(roff + j * q, q)],
                    send_sem=send_sems.at[sem0 + j],
                    recv_sem=recv_sems.at[sem0 + j],
                    device_id=tgt,
                    device_id_type=pl.DeviceIdType.MESH,
                )
                d.start()
                descs.append(d)
            return descs

        def finish_xy_rs(descs, base, q, col, roff, rbuf):
            for d in descs:
                d.wait()
            keep = base + fme * q
            out_ref[pl.ds(keep, q), pl.ds(col, NCOL)] = (
                out_ref[pl.ds(keep, q), pl.ds(col, NCOL)]
                + rbuf[pl.ds(roff, q), :]
                + rbuf[pl.ds(roff + q, q), :]
                + rbuf[pl.ds(roff + 2 * q, q), :]
            )
            return keep

        def start_exch(src_off, sz, col, roff, sem, rbuf, tgt):
            d = pltpu.make_async_remote_copy(
                src_ref=out_ref.at[pl.ds(src_off, sz), pl.ds(col, NCOL)],
                dst_ref=rbuf.at[pl.ds(roff, sz)],
                send_sem=send_sems.at[sem],
                recv_sem=recv_sems.at[sem],
                device_id=tgt,
                device_id_type=pl.DeviceIdType.MESH,
            )
            d.start()
            return d

        def finish_exch(d, dst_off, sz, col, roff, rbuf):
            d.wait()
            out_ref[pl.ds(dst_off, sz), pl.ds(col, NCOL)] = (
                out_ref[pl.ds(dst_off, sz), pl.ds(col, NCOL)]
                + rbuf[pl.ds(roff, sz), :]
            )

        def start_ag(base, cur, col, sem0, tgts):
            descs = []
            for j, tgt in enumerate(tgts):
                d = pltpu.make_async_remote_copy(
                    src_ref=out_ref.at[pl.ds(base, cur), pl.ds(col, NCOL)],
                    dst_ref=out_ref.at[pl.ds(base, cur), pl.ds(col, NCOL)],
                    send_sem=send_sems.at[sem0 + j],
                    recv_sem=recv_sems.at[sem0 + j],
                    device_id=tgt,
                    device_id_type=pl.DeviceIdType.MESH,
                )
                d.start()
                descs.append(d)
            return descs

        def finish_ag(descs):
            for d in descs:
                d.wait()

        dA = start_xy_rs(0, 256, 0, 0, 0, rbuf_a)
        dB = start_exch(0 + (1 - b1) * 512, 512, NCOL, 0, 10, rbuf_b, z1p)
        base_a = finish_xy_rs(dA, 0, 256, 0, 0, rbuf_a)
        base_b = b1 * 512
        finish_exch(dB, base_b, 512, NCOL, 0, rbuf_b)

        dA = start_exch(base_a + (1 - b1) * 128, 128, 0, 768, 3, rbuf_a, z1p)
        dB = start_xy_rs(base_b, 128, NCOL, 512, 11, rbuf_b)
        base_a = base_a + b1 * 128
        finish_exch(dA, base_a, 128, 0, 768, rbuf_a)
        base_b = finish_xy_rs(dB, base_b, 128, NCOL, 512, rbuf_b)

        dA = start_exch(base_a, 128, 0, 896, 4, rbuf_a, z2p)
        dB = start_exch(base_b, 128, NCOL, 896, 14, rbuf_b, z2p)
        finish_exch(dA, base_a, 128, 0, 896, rbuf_a)
        finish_exch(dB, base_b, 128, NCOL, 896, rbuf_b)

        dA = start_ag(base_a, 128, 0, 5, [z1p])
        dB = start_ag(base_b, 128, NCOL, 15, xy_tgts_only)
        finish_ag(dA)
        finish_ag(dB)
        base_a = base_a - b1 * 128
        base_b = base_b - fme * 128

        dA = start_ag(base_a, 256, 0, 6, xy_tgts_only)
        dB = start_ag(base_b, 512, NCOL, 19, [z1p])
        finish_ag(dA)
        finish_ag(dB)

    return pl.pallas_call(
        body,
        out_shape=jax.ShapeDtypeStruct((M, N), jnp.bfloat16),
        in_specs=[pl.BlockSpec(memory_space=pltpu.VMEM)],
        out_specs=pl.BlockSpec(memory_space=pltpu.VMEM),
        scratch_shapes=[
            pltpu.VMEM((1024, NCOL), jnp.bfloat16),
            pltpu.VMEM((1024, NCOL), jnp.bfloat16),
            pltpu.SemaphoreType.DMA((20,)),
            pltpu.SemaphoreType.DMA((20,)),
        ],
        compiler_params=pltpu.CompilerParams(collective_id=0),
    )(x2)
